# baseline (speedup 1.0000x reference)
import jax
import jax.numpy as jnp
from jax import lax
from jax.experimental import pallas as pl
from jax.experimental.pallas import tpu as pltpu
from jax.experimental.pallas import tpu_sc as plsc

BATCH = 16384
DIM = 32
VOCAB = 1000000
GROUP = 128 // DIM
NUM_WORKERS = 32
ROWS_PER_WORKER = BATCH // NUM_WORKERS
HALF = ROWS_PER_WORKER // 2
LANES = 16
CHUNKS_PER_HALF = HALF // LANES
DMA_ROWS = 128
DMAS_PER_HALF = HALF // DMA_ROWS


def _body(uidx_hbm, pidx_hbm, nidx_hbm, utab_hbm, itab_hbm, out_hbm,
          idx2d_v, uidx_v, pidx_v, nidx_v, ugrp_v, pgrp_v, ngrp_v,
          urows_v, prows_v, nrows_v, outbuf_v, *sems):
    cid = lax.axis_index("c")
    sid = lax.axis_index("s")
    wid = sid * 2 + cid
    base = wid * ROWS_PER_WORKER

    col0 = jnp.zeros((LANES,), jnp.int32)
    col1 = jnp.ones((LANES,), jnp.int32)
    lane_iota = lax.iota(jnp.int32, LANES)

    row_sl = pl.ds(base, ROWS_PER_WORKER)
    for src, flat, grp in ((uidx_hbm, uidx_v, ugrp_v),
                           (pidx_hbm, pidx_v, pgrp_v),
                           (nidx_hbm, nidx_v, ngrp_v)):
        pltpu.sync_copy(src.at[row_sl], idx2d_v)
        for c in range(ROWS_PER_WORKER // LANES):
            vals = plsc.load_gather(idx2d_v, [c * LANES + lane_iota, col0])
            flat[pl.ds(c * LANES, LANES)] = vals
            grp[pl.ds(c * LANES, LANES)] = lax.shift_right_logical(vals, 2)

    tables = ((uidx_v, ugrp_v, urows_v, utab_hbm),
              (pidx_v, pgrp_v, prows_v, itab_hbm),
              (nidx_v, ngrp_v, nrows_v, itab_hbm))

    def fire_half(half):
        hs = []
        for _, grp, rows, tab in tables:
            for k in range(DMAS_PER_HALF):
                isl = pl.ds(half * HALF + k * DMA_ROWS, DMA_ROWS)
                dsl = pl.ds(k * DMA_ROWS, DMA_ROWS)
                hs.append(pltpu.async_copy(
                    tab.at[grp.at[isl]], rows.at[dsl], sems[half]))
        return hs

    def dot_half(half):
        for c in range(CHUNKS_PER_HALF):
            slots = c * LANES + lane_iota
            accp = jnp.zeros((LANES,), jnp.float32)
            accn = jnp.zeros((LANES,), jnp.float32)
            offs = []
            for flat, _, _, _ in tables:
                raw = flat[pl.ds(half * HALF + c * LANES, LANES)]
                offs.append((raw & (GROUP - 1)) * DIM)
            uoff, poff, noff = offs
            for d in range(DIM):
                u = plsc.load_gather(urows_v, [slots, uoff + d])
                p = plsc.load_gather(prows_v, [slots, poff + d])
                n = plsc.load_gather(nrows_v, [slots, noff + d])
                accp = accp + u * p
                accn = accn + u * n
            lanes = half * HALF + c * LANES + lane_iota
            plsc.store_scatter(outbuf_v, [lanes, col0], accp)
            plsc.store_scatter(outbuf_v, [lanes, col1], accn)

    for half in range(2):
        for h in fire_half(half):
            h.wait()
        dot_half(half)

    pltpu.sync_copy(outbuf_v, out_hbm.at[pl.ds(base, ROWS_PER_WORKER)])


def kernel(user_inputs, pos_inputs, neg_inputs, user_table, item_table):
    mesh = plsc.VectorSubcoreMesh(core_axis_name="c", subcore_axis_name="s")
    run = pl.kernel(
        _body,
        out_type=jax.ShapeDtypeStruct((BATCH, 2), jnp.float32),
        mesh=mesh,
        scratch_types=[
            pltpu.VMEM((ROWS_PER_WORKER, 1), jnp.int32),
            pltpu.VMEM((ROWS_PER_WORKER,), jnp.int32),
            pltpu.VMEM((ROWS_PER_WORKER,), jnp.int32),
            pltpu.VMEM((ROWS_PER_WORKER,), jnp.int32),
            pltpu.VMEM((ROWS_PER_WORKER,), jnp.int32),
            pltpu.VMEM((ROWS_PER_WORKER,), jnp.int32),
            pltpu.VMEM((ROWS_PER_WORKER,), jnp.int32),
            pltpu.VMEM((HALF, 128), jnp.float32),
            pltpu.VMEM((HALF, 128), jnp.float32),
            pltpu.VMEM((HALF, 128), jnp.float32),
            pltpu.VMEM((ROWS_PER_WORKER, 2), jnp.float32),
        ] + [pltpu.SemaphoreType.DMA] * 2,
        compiler_params=pltpu.CompilerParams(
            needs_layout_passes=False,
            use_tc_tiling_on_sc=False,
        ),
    )
    ugrp = jnp.concatenate([user_table[k::GROUP] for k in range(GROUP)], axis=1)
    igrp = jnp.concatenate([item_table[k::GROUP] for k in range(GROUP)], axis=1)
    return run(user_inputs, pos_inputs, neg_inputs, ugrp, igrp)


# speedup vs baseline: 9.7952x; 9.7952x over previous
import jax
import jax.numpy as jnp
from jax import lax
from jax.experimental import pallas as pl
from jax.experimental.pallas import tpu as pltpu
from jax.experimental.pallas import tpu_sc as plsc

BATCH = 16384
DIM = 32
NUM_WORKERS = 32
ROWS_PER_WORKER = BATCH // NUM_WORKERS
IDX_CHUNK = 128
NUM_CHUNKS = ROWS_PER_WORKER // IDX_CHUNK
LANES = 16


def _body(uidx_hbm, pidx_hbm, nidx_hbm, user_table, item_table, out_hbm,
          uidx_v, pidx_v, nidx_v, urows_v, prows_v, nrows_v, outbuf_v,
          *sems):
    cid = lax.axis_index("c")
    sid = lax.axis_index("s")
    wid = sid * 2 + cid
    base = wid * ROWS_PER_WORKER

    pltpu.sync_copy(uidx_hbm.at[wid], uidx_v)
    pltpu.sync_copy(pidx_hbm.at[wid], pidx_v)
    pltpu.sync_copy(nidx_hbm.at[wid], nidx_v)

    handles = []
    for j in range(NUM_CHUNKS):
        sl = pl.ds(j * IDX_CHUNK, IDX_CHUNK)
        handles.append((
            pltpu.async_copy(user_table.at[uidx_v.at[j]], urows_v.at[sl], sems[j]),
            pltpu.async_copy(item_table.at[pidx_v.at[j]], prows_v.at[sl], sems[j]),
            pltpu.async_copy(item_table.at[nidx_v.at[j]], nrows_v.at[sl], sems[j]),
        ))

    col0 = jnp.zeros((LANES,), jnp.int32)
    col1 = jnp.ones((LANES,), jnp.int32)
    lane_iota = lax.iota(jnp.int32, LANES)

    for j in range(NUM_CHUNKS):
        for h in handles[j]:
            h.wait()

        def chunk(cc, carry, j=j):
            lanes = j * IDX_CHUNK + cc * LANES + lane_iota
            accp = jnp.zeros((LANES,), jnp.float32)
            accn = jnp.zeros((LANES,), jnp.float32)
            for d in range(DIM):
                dvec = jnp.full((LANES,), d, jnp.int32)
                u = plsc.load_gather(urows_v, [lanes, dvec])
                p = plsc.load_gather(prows_v, [lanes, dvec])
                n = plsc.load_gather(nrows_v, [lanes, dvec])
                accp = accp + u * p
                accn = accn + u * n
            plsc.store_scatter(outbuf_v, [lanes, col0], accp)
            plsc.store_scatter(outbuf_v, [lanes, col1], accn)
            return carry

        lax.fori_loop(0, IDX_CHUNK // LANES, chunk, 0)

    pltpu.sync_copy(outbuf_v, out_hbm.at[pl.ds(base, ROWS_PER_WORKER)])


def kernel(user_inputs, pos_inputs, neg_inputs, user_table, item_table):
    mesh = plsc.VectorSubcoreMesh(core_axis_name="c", subcore_axis_name="s")
    run = pl.kernel(
        _body,
        out_type=jax.ShapeDtypeStruct((BATCH, 2), jnp.float32),
        mesh=mesh,
        scratch_types=[
            pltpu.VMEM((NUM_CHUNKS, IDX_CHUNK), jnp.int32),
            pltpu.VMEM((NUM_CHUNKS, IDX_CHUNK), jnp.int32),
            pltpu.VMEM((NUM_CHUNKS, IDX_CHUNK), jnp.int32),
            pltpu.VMEM((ROWS_PER_WORKER, DIM), jnp.float32),
            pltpu.VMEM((ROWS_PER_WORKER, DIM), jnp.float32),
            pltpu.VMEM((ROWS_PER_WORKER, DIM), jnp.float32),
            pltpu.VMEM((ROWS_PER_WORKER, 2), jnp.float32),
        ] + [pltpu.SemaphoreType.DMA] * NUM_CHUNKS,
        compiler_params=pltpu.CompilerParams(
            needs_layout_passes=False,
            use_tc_tiling_on_sc=False,
        ),
    )
    shape = (NUM_WORKERS, NUM_CHUNKS, IDX_CHUNK)
    return run(
        user_inputs.reshape(shape),
        pos_inputs.reshape(shape),
        neg_inputs.reshape(shape),
        user_table,
        item_table,
    )
